# baseline (device time: 82196 ns/iter reference)
import jax
import jax.numpy as jnp
from jax import lax
from jax.experimental import pallas as pl
from jax.experimental.pallas import tpu as pltpu

B, S, H, Dh, Dr = 2, 512, 16, 128, 32
D = 2048
DC = 128
N_X = 2
N_DEV = 4
HG = H // N_DEV
GW = HG * Dh
GWR = HG * Dr
BS = B * S
SCALE = (Dh + Dr) ** -0.5
BF16 = jnp.bfloat16
F32 = jnp.float32

_VMEM = pl.BlockSpec(memory_space=pltpu.VMEM)
_MESH = pl.DeviceIdType.MESH


def _proj_body(x_ref, wq_ref, wqr_ref, wkr_ref, wdkv_ref,
               wuk_mine_ref, wuk_send_ref, wuv_mine_ref, wuv_send_ref,
               q_ref, qr_ref, kr_ref, kg_ref, vg_ref,
               cg, wukg, wuvg, wuk_sb, wuv_sb, send_sems, recv_sems):
    my_x = lax.axis_index("x")
    my_y = lax.axis_index("y")
    xpeer = (1 - my_x, my_y)

    barrier = pltpu.get_barrier_semaphore()
    pl.semaphore_signal(barrier, inc=1, device_id=xpeer, device_id_type=_MESH)
    pl.semaphore_wait(barrier, 1)

    wukg[my_x] = wuk_mine_ref[...].astype(BF16)
    wuvg[my_x] = wuv_mine_ref[...].astype(BF16)
    wuk_sb[...] = wuk_send_ref[...].astype(BF16)
    wuv_sb[...] = wuv_send_ref[...].astype(BF16)

    rdmas = []
    for i, (src, dst) in enumerate((
            (wuk_sb, wukg.at[my_x]),
            (wuv_sb, wuvg.at[my_x]),
    )):
        rdma = pltpu.make_async_remote_copy(
            src_ref=src, dst_ref=dst,
            send_sem=send_sems.at[i], recv_sem=recv_sems.at[i],
            device_id=xpeer, device_id_type=_MESH,
        )
        rdma.start()
        rdmas.append(rdma)

    xb = x_ref[...].astype(BF16)
    cg[my_x] = jnp.dot(xb, wdkv_ref[...].astype(BF16),
                       preferred_element_type=F32).astype(BF16)
    rdma_c = pltpu.make_async_remote_copy(
        src_ref=cg.at[my_x], dst_ref=cg.at[my_x],
        send_sem=send_sems.at[2], recv_sem=recv_sems.at[2],
        device_id=xpeer, device_id_type=_MESH,
    )
    rdma_c.start()
    rdmas.append(rdma_c)

    q_ref[...] = jnp.dot(xb, wq_ref[...].astype(BF16),
                         preferred_element_type=F32).astype(BF16)
    qr_ref[...] = jnp.dot(xb, wqr_ref[...].astype(BF16),
                          preferred_element_type=F32).astype(BF16)
    kr_ref[...] = jnp.dot(xb, wkr_ref[...].astype(BF16),
                          preferred_element_type=F32).astype(BF16)

    for rdma in rdmas:
        rdma.wait()

    kg_ref[...] = (jnp.dot(cg[0], wukg[0], preferred_element_type=F32)
                   + jnp.dot(cg[1], wukg[1],
                             preferred_element_type=F32)).astype(BF16)
    vg_ref[...] = (jnp.dot(cg[0], wuvg[0], preferred_element_type=F32)
                   + jnp.dot(cg[1], wuvg[1],
                             preferred_element_type=F32)).astype(BF16)


def _attn_out_body(q_ref, k_ref, v_ref, qr_ref, kr_ref, wo_ref,
                   out_ref, og, wob, send_sems, recv_sems):
    my_x = lax.axis_index("x")
    my_y = lax.axis_index("y")
    g = 2 * my_x + my_y
    gx = 2 * (1 - my_x) + my_y
    gy = 2 * my_x + (1 - my_y)
    gd = 2 * (1 - my_x) + (1 - my_y)
    peers = ((1 - my_x, my_y), (my_x, 1 - my_y), (1 - my_x, 1 - my_y))

    barrier = pltpu.get_barrier_semaphore()
    for p_id in peers:
        pl.semaphore_signal(barrier, inc=1, device_id=p_id,
                            device_id_type=_MESH)
    pl.semaphore_wait(barrier, 3)

    rdmas = []
    for h in range(HG):
        for b in range(B):
            rows = pl.ds(b * S, S)
            q = q_ref[rows, h * Dh:(h + 1) * Dh]
            k = k_ref[rows, h * Dh:(h + 1) * Dh]
            qr = qr_ref[rows, h * Dr:(h + 1) * Dr]
            s = lax.dot_general(q, k, (((1,), (1,)), ((), ())),
                                preferred_element_type=F32)
            s += lax.dot_general(qr, kr_ref[rows, :],
                                 (((1,), (1,)), ((), ())),
                                 preferred_element_type=F32)
            s *= SCALE
            m = jnp.max(s, axis=1, keepdims=True)
            e = jnp.exp(s - m)
            p = (e / jnp.sum(e, axis=1, keepdims=True)).astype(BF16)
            og[g, rows, h * Dh:(h + 1) * Dh] = jnp.dot(
                p, v_ref[rows, h * Dh:(h + 1) * Dh],
                preferred_element_type=F32).astype(BF16)
        for i, p_id in enumerate(peers):
            rdma = pltpu.make_async_remote_copy(
                src_ref=og.at[g, :, pl.ds(h * Dh, Dh)],
                dst_ref=og.at[g, :, pl.ds(h * Dh, Dh)],
                send_sem=send_sems.at[h * 3 + i],
                recv_sem=recv_sems.at[h * 3 + i],
                device_id=p_id, device_id_type=_MESH,
            )
            rdma.start()
            rdmas.append(rdma)

    wob[...] = wo_ref[...].astype(BF16)

    out_ref[...] = jnp.dot(og[g], wob[pl.ds(g * GW, GW), :],
                           preferred_element_type=F32)
    for i, slot in enumerate((gx, gy, gd)):
        for h in range(HG):
            rdmas[h * 3 + i].wait_recv()
        out_ref[...] += jnp.dot(og[slot], wob[pl.ds(slot * GW, GW), :],
                                preferred_element_type=F32)
    for rdma in rdmas:
        rdma.wait_send()


def kernel(x, Wdkv, Wuk, Wuv, Wq, Wqr, Wkr, Wo):
    xb = x.reshape(BS, D)
    gx_ = lax.axis_index("x")
    gy_ = lax.axis_index("y")
    g = 2 * gx_ + gy_
    peer_g = 2 * (1 - gx_) + gy_

    wq_g = lax.dynamic_slice(Wq, (0, g * GW), (D, GW))
    wqr_g = lax.dynamic_slice(Wqr, (0, g * GWR), (D, GWR))
    wuk_mine = lax.dynamic_slice(Wuk, (0, g * GW), (DC, GW))
    wuk_send = lax.dynamic_slice(Wuk, (0, peer_g * GW), (DC, GW))
    wuv_mine = lax.dynamic_slice(Wuv, (0, g * GW), (DC, GW))
    wuv_send = lax.dynamic_slice(Wuv, (0, peer_g * GW), (DC, GW))

    q, qr, kr, kg, vg = pl.pallas_call(
        _proj_body,
        out_shape=(
            jax.ShapeDtypeStruct((BS, GW), BF16),
            jax.ShapeDtypeStruct((BS, GWR), BF16),
            jax.ShapeDtypeStruct((BS, Dr), BF16),
            jax.ShapeDtypeStruct((BS, GW), BF16),
            jax.ShapeDtypeStruct((BS, GW), BF16),
        ),
        in_specs=[_VMEM] * 9,
        out_specs=(_VMEM,) * 5,
        scratch_shapes=[
            pltpu.VMEM((N_X, BS, DC), BF16),
            pltpu.VMEM((N_X, DC, GW), BF16),
            pltpu.VMEM((N_X, DC, GW), BF16),
            pltpu.VMEM((DC, GW), BF16),
            pltpu.VMEM((DC, GW), BF16),
            pltpu.SemaphoreType.DMA((3,)),
            pltpu.SemaphoreType.DMA((3,)),
        ],
        compiler_params=pltpu.CompilerParams(
            collective_id=0, vmem_limit_bytes=100 * 1024 * 1024),
    )(xb, wq_g, wqr_g, Wkr, Wdkv, wuk_mine, wuk_send, wuv_mine, wuv_send)

    out = pl.pallas_call(
        _attn_out_body,
        out_shape=jax.ShapeDtypeStruct((BS, D), F32),
        in_specs=[_VMEM] * 6,
        out_specs=_VMEM,
        scratch_shapes=[
            pltpu.VMEM((N_DEV, BS, GW), BF16),
            pltpu.VMEM((D, D), BF16),
            pltpu.SemaphoreType.DMA((HG * 3,)),
            pltpu.SemaphoreType.DMA((HG * 3,)),
        ],
        compiler_params=pltpu.CompilerParams(
            collective_id=1, vmem_limit_bytes=100 * 1024 * 1024),
    )(q, kg, vg, qr, kr, Wo)
    return out.reshape(B, S, D)


# device time: 65348 ns/iter; 1.2578x vs baseline; 1.2578x over previous
import jax
import jax.numpy as jnp
from jax import lax
from jax.experimental import pallas as pl
from jax.experimental.pallas import tpu as pltpu

B, S, H, Dh, Dr = 2, 512, 16, 128, 32
D = 2048
DC = 128
N_X = 2
N_DEV = 4
HG = H // N_DEV
GW = HG * Dh
GWR = HG * Dr
BS = B * S
SCALE = (Dh + Dr) ** -0.5
BF16 = jnp.bfloat16
F32 = jnp.float32

_VMEM = pl.BlockSpec(memory_space=pltpu.VMEM)
_MESH = pl.DeviceIdType.MESH


def _proj_body(x_ref, wq_ref, wqr_ref, wkr_ref, wdkv_ref,
               wuk_mine_ref, wuk_send_ref, wuv_mine_ref, wuv_send_ref,
               q_ref, qr_ref, kr_ref, kg_ref, vg_ref,
               cg, wukg, wuvg, wuk_sb, wuv_sb, send_sems, recv_sems):
    my_x = lax.axis_index("x")
    my_y = lax.axis_index("y")
    xpeer = (1 - my_x, my_y)

    barrier = pltpu.get_barrier_semaphore()
    pl.semaphore_signal(barrier, inc=1, device_id=xpeer, device_id_type=_MESH)
    pl.semaphore_wait(barrier, 1)

    xb = x_ref[...].astype(BF16)
    cg[my_x] = jnp.dot(xb, wdkv_ref[...].astype(BF16),
                       preferred_element_type=F32).astype(BF16)
    wukg[my_x] = wuk_mine_ref[...].astype(BF16)
    wuvg[my_x] = wuv_mine_ref[...].astype(BF16)
    wuk_sb[...] = wuk_send_ref[...].astype(BF16)
    wuv_sb[...] = wuv_send_ref[...].astype(BF16)

    rdmas = []
    for i, (src, dst) in enumerate((
            (cg.at[my_x], cg.at[my_x]),
            (wuk_sb, wukg.at[my_x]),
            (wuv_sb, wuvg.at[my_x]),
    )):
        rdma = pltpu.make_async_remote_copy(
            src_ref=src, dst_ref=dst,
            send_sem=send_sems.at[i], recv_sem=recv_sems.at[i],
            device_id=xpeer, device_id_type=_MESH,
        )
        rdma.start()
        rdmas.append(rdma)

    q_ref[...] = jnp.dot(xb, wq_ref[...].astype(BF16),
                         preferred_element_type=F32).astype(BF16)
    qr_ref[...] = jnp.dot(xb, wqr_ref[...].astype(BF16),
                          preferred_element_type=F32).astype(BF16)
    kr_ref[...] = jnp.dot(xb, wkr_ref[...].astype(BF16),
                          preferred_element_type=F32).astype(BF16)

    for rdma in rdmas:
        rdma.wait()

    kg_ref[...] = (jnp.dot(cg[0], wukg[0], preferred_element_type=F32)
                   + jnp.dot(cg[1], wukg[1],
                             preferred_element_type=F32)).astype(BF16)
    vg_ref[...] = (jnp.dot(cg[0], wuvg[0], preferred_element_type=F32)
                   + jnp.dot(cg[1], wuvg[1],
                             preferred_element_type=F32)).astype(BF16)


def _attn_out_body(q_ref, k_ref, v_ref, qr_ref, kr_ref, wo_ref,
                   out_ref, og, send_sems, recv_sems):
    my_x = lax.axis_index("x")
    my_y = lax.axis_index("y")
    g = 2 * my_x + my_y
    gx = 2 * (1 - my_x) + my_y
    gy = 2 * my_x + (1 - my_y)
    gd = 2 * (1 - my_x) + (1 - my_y)
    peers = ((1 - my_x, my_y), (my_x, 1 - my_y), (1 - my_x, 1 - my_y))

    barrier = pltpu.get_barrier_semaphore()
    for p_id in peers:
        pl.semaphore_signal(barrier, inc=1, device_id=p_id,
                            device_id_type=_MESH)
    pl.semaphore_wait(barrier, 3)

    rdmas = []
    for h in range(HG):
        for b in range(B):
            rows = pl.ds(b * S, S)
            q = q_ref[rows, h * Dh:(h + 1) * Dh]
            k = k_ref[rows, h * Dh:(h + 1) * Dh]
            qr = qr_ref[rows, h * Dr:(h + 1) * Dr]
            s = lax.dot_general(q, k, (((1,), (1,)), ((), ())),
                                preferred_element_type=F32)
            s += lax.dot_general(qr, kr_ref[rows, :],
                                 (((1,), (1,)), ((), ())),
                                 preferred_element_type=F32)
            s *= SCALE
            m = jnp.max(s, axis=1, keepdims=True)
            e = jnp.exp(s - m)
            p = (e / jnp.sum(e, axis=1, keepdims=True)).astype(BF16)
            og[g, rows, h * Dh:(h + 1) * Dh] = jnp.dot(
                p, v_ref[rows, h * Dh:(h + 1) * Dh],
                preferred_element_type=F32).astype(BF16)
        for i, p_id in enumerate(peers):
            rdma = pltpu.make_async_remote_copy(
                src_ref=og.at[g, :, pl.ds(h * Dh, Dh)],
                dst_ref=og.at[g, :, pl.ds(h * Dh, Dh)],
                send_sem=send_sems.at[h * 3 + i],
                recv_sem=recv_sems.at[h * 3 + i],
                device_id=p_id, device_id_type=_MESH,
            )
            rdma.start()
            rdmas.append(rdma)

    def wo_slice(q_):
        return wo_ref[pl.ds(q_ * GW, GW), :].astype(BF16)

    out_ref[...] = jnp.dot(og[g], wo_slice(g), preferred_element_type=F32)
    for i, slot in enumerate((gx, gy, gd)):
        for h in range(HG):
            rdmas[h * 3 + i].wait_recv()
        out_ref[...] += jnp.dot(og[slot], wo_slice(slot),
                                preferred_element_type=F32)
    for rdma in rdmas:
        rdma.wait_send()


def kernel(x, Wdkv, Wuk, Wuv, Wq, Wqr, Wkr, Wo):
    xb = x.reshape(BS, D)
    gx_ = lax.axis_index("x")
    gy_ = lax.axis_index("y")
    g = 2 * gx_ + gy_
    peer_g = 2 * (1 - gx_) + gy_

    wq_g = lax.dynamic_slice(Wq, (0, g * GW), (D, GW))
    wqr_g = lax.dynamic_slice(Wqr, (0, g * GWR), (D, GWR))
    wuk_mine = lax.dynamic_slice(Wuk, (0, g * GW), (DC, GW))
    wuk_send = lax.dynamic_slice(Wuk, (0, peer_g * GW), (DC, GW))
    wuv_mine = lax.dynamic_slice(Wuv, (0, g * GW), (DC, GW))
    wuv_send = lax.dynamic_slice(Wuv, (0, peer_g * GW), (DC, GW))

    q, qr, kr, kg, vg = pl.pallas_call(
        _proj_body,
        out_shape=(
            jax.ShapeDtypeStruct((BS, GW), BF16),
            jax.ShapeDtypeStruct((BS, GWR), BF16),
            jax.ShapeDtypeStruct((BS, Dr), BF16),
            jax.ShapeDtypeStruct((BS, GW), BF16),
            jax.ShapeDtypeStruct((BS, GW), BF16),
        ),
        in_specs=[_VMEM] * 9,
        out_specs=(_VMEM,) * 5,
        scratch_shapes=[
            pltpu.VMEM((N_X, BS, DC), BF16),
            pltpu.VMEM((N_X, DC, GW), BF16),
            pltpu.VMEM((N_X, DC, GW), BF16),
            pltpu.VMEM((DC, GW), BF16),
            pltpu.VMEM((DC, GW), BF16),
            pltpu.SemaphoreType.DMA((3,)),
            pltpu.SemaphoreType.DMA((3,)),
        ],
        compiler_params=pltpu.CompilerParams(collective_id=0),
    )(xb, wq_g, wqr_g, Wkr, Wdkv, wuk_mine, wuk_send, wuv_mine, wuv_send)

    out = pl.pallas_call(
        _attn_out_body,
        out_shape=jax.ShapeDtypeStruct((BS, D), F32),
        in_specs=[_VMEM] * 6,
        out_specs=_VMEM,
        scratch_shapes=[
            pltpu.VMEM((N_DEV, BS, GW), BF16),
            pltpu.SemaphoreType.DMA((HG * 3,)),
            pltpu.SemaphoreType.DMA((HG * 3,)),
        ],
        compiler_params=pltpu.CompilerParams(collective_id=1),
    )(q, kg, vg, qr, kr, Wo)
    return out.reshape(B, S, D)
